# Initial kernel scaffold; baseline (speedup 1.0000x reference)
import jax
import jax.numpy as jnp
from jax.experimental import pallas as pl


def kernel(x, ln_g, ln_b, Wr, br, W1, b1, W2, b2):
    raise NotImplementedError("write your pallas kernel here")


# speedup vs baseline: 3.0858x; 3.0858x over previous
import functools

import jax
import jax.numpy as jnp
from jax.experimental import pallas as pl
from jax.experimental.pallas import tpu as pltpu


def _gelu_exact(x):
    return 0.5 * x * (1.0 + jax.lax.erf(x * 0.7071067811865475))


def _moe_body(x_ref, g_ref, b_ref, wr_ref, br_ref, w1_ref, b1_ref, w2_ref,
              b2_ref, out_ref, h_s, w_s, *, n_experts):
    e = pl.program_id(0)
    f = pl.program_id(1)

    @pl.when((e == 0) & (f == 0))
    def _init():
        xv = x_ref[...]
        mu = jnp.mean(xv, axis=-1, keepdims=True)
        xc = xv - mu
        var = jnp.mean(xc * xc, axis=-1, keepdims=True)
        h = xc * jax.lax.rsqrt(var + 1e-5) * g_ref[0, :] + b_ref[0, :]
        logits = jnp.dot(h, wr_ref[...],
                         preferred_element_type=jnp.float32) + br_ref[0, :]
        iota = jax.lax.broadcasted_iota(jnp.int32, logits.shape, 1)
        v1 = jnp.max(logits, axis=-1, keepdims=True)
        i1 = jnp.min(jnp.where(logits >= v1, iota, n_experts),
                     axis=-1, keepdims=True)
        mk1 = iota == i1
        l2 = jnp.where(mk1, jnp.float32(-1e30), logits)
        v2 = jnp.max(l2, axis=-1, keepdims=True)
        i2 = jnp.min(jnp.where(l2 >= v2, iota, n_experts),
                     axis=-1, keepdims=True)
        mk2 = iota == i2
        g1 = 1.0 / (1.0 + jnp.exp(v2 - v1))
        w = jnp.where(mk1, g1, 0.0) + jnp.where(mk2, 1.0 - g1, 0.0)
        w_s[...] = w
        h_s[...] = h.astype(jnp.bfloat16)
        out_ref[...] = xv + jnp.dot(w, b2_ref[...],
                                    preferred_element_type=jnp.float32)

    h = h_s[...]
    hid = jnp.dot(h, w1_ref[0], preferred_element_type=jnp.float32)
    hid = hid + b1_ref[0, 0, :]
    hid = _gelu_exact(hid)
    iota_e = jax.lax.broadcasted_iota(jnp.int32, w_s.shape, 1)
    wcol = jnp.sum(jnp.where(iota_e == e, w_s[...], 0.0),
                   axis=-1, keepdims=True)
    hid = (hid * wcol).astype(jnp.bfloat16)
    out_ref[...] += jnp.dot(hid, w2_ref[0],
                            preferred_element_type=jnp.float32)


def kernel(x, ln_g, ln_b, Wr, br, W1, b1, W2, b2):
    B, T, D = x.shape
    E = Wr.shape[1]
    DFF = W1.shape[2]
    N = B * T
    TILE_F = 768
    n_f = DFF // TILE_F

    x2 = x.reshape(N, D)
    w1b = W1.astype(jnp.bfloat16)
    w2b = W2.astype(jnp.bfloat16)

    out = pl.pallas_call(
        functools.partial(_moe_body, n_experts=E),
        grid=(E, n_f),
        in_specs=[
            pl.BlockSpec((N, D), lambda e, f: (0, 0)),
            pl.BlockSpec((1, D), lambda e, f: (0, 0)),
            pl.BlockSpec((1, D), lambda e, f: (0, 0)),
            pl.BlockSpec((D, E), lambda e, f: (0, 0)),
            pl.BlockSpec((1, E), lambda e, f: (0, 0)),
            pl.BlockSpec((1, D, TILE_F), lambda e, f: (e, 0, f)),
            pl.BlockSpec((1, 1, TILE_F), lambda e, f: (e, 0, f)),
            pl.BlockSpec((1, TILE_F, D), lambda e, f: (e, f, 0)),
            pl.BlockSpec((E, D), lambda e, f: (0, 0)),
        ],
        out_specs=pl.BlockSpec((N, D), lambda e, f: (0, 0)),
        out_shape=jax.ShapeDtypeStruct((N, D), jnp.float32),
        scratch_shapes=[
            pltpu.VMEM((N, D), jnp.bfloat16),
            pltpu.VMEM((N, E), jnp.float32),
        ],
        compiler_params=pltpu.CompilerParams(
            dimension_semantics=("arbitrary", "arbitrary"),
            vmem_limit_bytes=100 * 1024 * 1024,
        ),
    )(x2, ln_g.reshape(1, D), ln_b.reshape(1, D), Wr, br.reshape(1, E),
      w1b, b1.reshape(E, 1, DFF), w2b, b2)
    return out.reshape(B, T, D)
